# baseline (speedup 1.0000x reference)
import functools

import jax
import jax.numpy as jnp
from jax import lax
from jax.experimental import pallas as pl
from jax.experimental.pallas import tpu as pltpu
from jax.experimental.pallas import tpu_sc as plsc

SENT = 64
ANN = 256
FC7 = 2048
JEMB = 512
SB = 8
NLANE = 16
NCHUNK = ANN // NLANE

HI = jax.lax.Precision.HIGHEST


_GDN = lax.GatherDimensionNumbers(offset_dims=(), collapsed_slice_dims=(0,),
                                  start_index_map=(0,))


def _lane_allreduce(v, op):
    iota = lax.iota(jnp.int32, NLANE)
    for b in range(4):
        idx = jnp.bitwise_xor(iota, jnp.int32(1 << b)).reshape(NLANE, 1)
        v = op(v, lax.gather(v, idx, _GDN, (1,),
                             mode=lax.GatherScatterMode.PROMISE_IN_BOUNDS))
    return v


def _sc_kernel(attn_hbm, dist_t_hbm, dists_hbm, maxid_hbm,
               attn_v, row16_v, row_v, idx_v, idst_v, sem):
    cid = lax.axis_index("c")
    sid = lax.axis_index("s")
    w = sid * 2 + cid
    iota = lax.iota(jnp.int32, NLANE)
    for t in range(2):
        s = w * 2 + t
        pltpu.sync_copy(attn_hbm.at[pl.ds(s, 1)], attn_v)
        m = jnp.full((NLANE,), -jnp.inf, jnp.float32)
        for k in range(NCHUNK):
            m = jnp.maximum(m, attn_v[0, pl.ds(k * NLANE, NLANE)])
        msplat = _lane_allreduce(m, jnp.maximum)
        idxv = jnp.full((NLANE,), ANN, jnp.int32)
        for k in range(NCHUNK):
            ch = attn_v[0, pl.ds(k * NLANE, NLANE)]
            idxv = jnp.minimum(
                idxv, jnp.where(ch == msplat, iota + (k * NLANE), ANN))
        amax = _lane_allreduce(idxv, jnp.minimum)
        idst_v[0, :] = amax
        pltpu.sync_copy(idst_v, maxid_hbm.at[pl.ds(s, 1)])
        idx_v[...] = amax
        pltpu.async_copy(dist_t_hbm.at[idx_v], row16_v, sem).wait()
        okv = jnp.where(msplat == 0.0, 0.0, 1.0)
        addv = (1.0 - okv) * 100.0
        for k in range(NCHUNK):
            sl = pl.ds(k * NLANE, NLANE)
            row_v[0, sl] = row16_v[0, sl] * okv + addv
        pltpu.sync_copy(row_v, dists_hbm.at[pl.ds(s, 1)])


def _tc_kernel(attn, cxt_feats, fc_w, w7, lw, b2, cw,
               fuse, base_s, cf_s, w2_s):
    pid = pl.program_id(0)

    @pl.when(pid == 0)
    def _prologue():
        a = attn[...]
        m = jnp.max(a, axis=1, keepdims=True)
        cols = lax.broadcasted_iota(jnp.int32, (SENT, ANN), 1)
        ids = jnp.min(jnp.where(a == m, cols, ANN), axis=1,
                      keepdims=True)
        onehot = (cols == ids).astype(jnp.float32)
        ok = jnp.where(m == 0.0, 0.0, 1.0)

        w2_s[...] = jnp.transpose(fc_w[:, FC7:FC7 + 5] * lw[...]) \
            .reshape(5, JEMB)

        g = lax.dot(onehot, cw[...], precision=HI)
        ss = g[:, :ANN] * g[:, :ANN]
        for c in range(1, 5):
            lc = g[:, c * ANN:(c + 1) * ANN]
            ss = ss + lc * lc
        invl = ok / jnp.maximum(jnp.sqrt(ss), 1e-12)
        cf_s[...] = g * jnp.concatenate([invl] * 5, axis=1)

        rf = lax.dot(onehot, cxt_feats[...], precision=HI)
        n = jnp.sqrt(jnp.sum(rf * rf, axis=1, keepdims=True))
        inv7 = ok / jnp.maximum(n, 1e-12)
        rfn = rf * inv7 * w7[...]
        base_s[...] = lax.dot_general(rfn, fc_w[:, :FC7],
                                      (((1,), (1,)), ((), ()))) + b2[...]

    sl = pl.ds(pid * SB, SB)
    cf = cf_s[sl, :]
    w2 = w2_s[...]
    out = jnp.broadcast_to(base_s[sl, :][:, None, :], (SB, ANN, JEMB))
    for c in range(5):
        out = out + cf[:, c * ANN:(c + 1) * ANN][:, :, None] \
            * w2[c][None, None, :]
    fuse[...] = out


@functools.partial(jax.jit, static_argnames=("interpret",))
def _run(cxt_feats, cxt_lfeats, obj_attn, dist, fc7_norm_w, lfeat_norm_w,
         fc_w, fc_b, interpret=False):
    cw = jnp.transpose(cxt_lfeats, (1, 2, 0)).reshape(ANN, 5 * ANN)
    dist_t = jnp.transpose(dist.reshape(ANN, ANN), (1, 0))
    b2 = fc_b.reshape(1, JEMB)

    sc = pl.kernel(
        _sc_kernel,
        out_type=[
            jax.ShapeDtypeStruct((SENT, ANN), jnp.float32),
            jax.ShapeDtypeStruct((SENT, NLANE), jnp.int32),
        ],
        mesh=plsc.VectorSubcoreMesh(core_axis_name="c",
                                    subcore_axis_name="s"),
        scratch_types=[
            pltpu.VMEM((1, ANN), jnp.float32),
            pltpu.VMEM((NLANE, ANN), jnp.float32),
            pltpu.VMEM((1, ANN), jnp.float32),
            pltpu.VMEM((NLANE,), jnp.int32),
            pltpu.VMEM((1, NLANE), jnp.int32),
            pltpu.SemaphoreType.DMA,
        ],
    )
    dists, maxid16 = sc(obj_attn, dist_t)

    fuse, = pl.pallas_call(
        _tc_kernel,
        grid=(SENT // SB,),
        in_specs=[
            pl.BlockSpec((SENT, ANN), lambda i: (0, 0)),
            pl.BlockSpec((ANN, FC7), lambda i: (0, 0)),
            pl.BlockSpec((JEMB, FC7 + 5), lambda i: (0, 0)),
            pl.BlockSpec((1, FC7), lambda i: (0, 0)),
            pl.BlockSpec((1, 5), lambda i: (0, 0)),
            pl.BlockSpec((1, JEMB), lambda i: (0, 0)),
            pl.BlockSpec((ANN, 5 * ANN), lambda i: (0, 0)),
        ],
        out_specs=[
            pl.BlockSpec((SB, ANN, JEMB), lambda i: (i, 0, 0)),
        ],
        out_shape=[
            jax.ShapeDtypeStruct((SENT, ANN, JEMB), jnp.float32),
        ],
        scratch_shapes=[
            pltpu.VMEM((SENT, JEMB), jnp.float32),
            pltpu.VMEM((SENT, 5 * ANN), jnp.float32),
            pltpu.VMEM((5, JEMB), jnp.float32),
        ],
        interpret=interpret,
    )(obj_attn, cxt_feats, fc_w, fc7_norm_w, lfeat_norm_w, b2, cw)

    return fuse, dists, maxid16[:, 0]


def kernel(cxt_feats, cxt_lfeats, obj_attn, wo_obj_idx, dist,
           fc7_norm_w, lfeat_norm_w, fc_w, fc_b):
    del wo_obj_idx
    return _run(cxt_feats, cxt_lfeats, obj_attn, dist, fc7_norm_w,
                lfeat_norm_w, fc_w, fc_b)


# speedup vs baseline: 1.3997x; 1.3997x over previous
import functools

import jax
import jax.numpy as jnp
from jax.experimental import pallas as pl
from jax.experimental.pallas import tpu as pltpu

SENT = 64
ANN = 256
FC7 = 2048
JEMB = 512
SB = 8

HI = jax.lax.Precision.HIGHEST


def _kernel(attn, cxt_feats, dist2, fc_w, w7, lw, b2, cw,
            fuse, dists, maxid, base_s, cf_s, w2_s):
    pid = pl.program_id(0)

    @pl.when(pid == 0)
    def _prologue():
        a = attn[...]
        m = jnp.max(a, axis=1, keepdims=True)
        cols = jax.lax.broadcasted_iota(jnp.int32, (SENT, ANN), 1)
        ids = jnp.min(jnp.where(a == m, cols, ANN), axis=1,
                      keepdims=True)
        maxid[...] = ids
        onehot = (cols == ids).astype(jnp.float32)
        ok = jnp.where(m == 0.0, 0.0, 1.0)

        dg = jax.lax.dot_general(onehot, dist2[...],
                                 (((1,), (1,)), ((), ())), precision=HI)
        dists[...] = jnp.where(ok == 0.0, 100.0, dg)

        w2_s[...] = jnp.transpose(fc_w[:, FC7:FC7 + 5] * lw[...]) \
            .reshape(5, JEMB)

        g = jax.lax.dot(onehot, cw[...], precision=HI)
        ss = g[:, :ANN] * g[:, :ANN]
        for c in range(1, 5):
            lc = g[:, c * ANN:(c + 1) * ANN]
            ss = ss + lc * lc
        invl = ok / jnp.maximum(jnp.sqrt(ss), 1e-12)
        cf_s[...] = g * jnp.concatenate([invl] * 5, axis=1)

        rf = jax.lax.dot(onehot, cxt_feats[...], precision=HI)
        n = jnp.sqrt(jnp.sum(rf * rf, axis=1, keepdims=True))
        inv7 = ok / jnp.maximum(n, 1e-12)
        rfn = rf * inv7 * w7[...]
        base_s[...] = jax.lax.dot_general(rfn, fc_w[:, :FC7],
                                          (((1,), (1,)), ((), ()))) + b2[...]

    sl = pl.ds(pid * SB, SB)
    cf = cf_s[sl, :]
    w2 = w2_s[...]
    out = jnp.broadcast_to(base_s[sl, :][:, None, :], (SB, ANN, JEMB))
    for c in range(5):
        out = out + cf[:, c * ANN:(c + 1) * ANN][:, :, None] \
            * w2[c][None, None, :]
    fuse[...] = out


@functools.partial(jax.jit, static_argnames=("interpret",))
def _run(cxt_feats, cxt_lfeats, obj_attn, dist, fc7_norm_w, lfeat_norm_w,
         fc_w, fc_b, interpret=False):
    cw = jnp.transpose(cxt_lfeats, (1, 2, 0)).reshape(ANN, 5 * ANN)
    dist2 = dist.reshape(ANN, ANN)
    b2 = fc_b.reshape(1, JEMB)

    fuse, dists, maxid = pl.pallas_call(
        _kernel,
        grid=(SENT // SB,),
        in_specs=[
            pl.BlockSpec((SENT, ANN), lambda i: (0, 0)),
            pl.BlockSpec((ANN, FC7), lambda i: (0, 0)),
            pl.BlockSpec((ANN, ANN), lambda i: (0, 0)),
            pl.BlockSpec((JEMB, FC7 + 5), lambda i: (0, 0)),
            pl.BlockSpec((1, FC7), lambda i: (0, 0)),
            pl.BlockSpec((1, 5), lambda i: (0, 0)),
            pl.BlockSpec((1, JEMB), lambda i: (0, 0)),
            pl.BlockSpec((ANN, 5 * ANN), lambda i: (0, 0)),
        ],
        out_specs=[
            pl.BlockSpec((SB, ANN, JEMB), lambda i: (i, 0, 0)),
            pl.BlockSpec((SENT, ANN), lambda i: (0, 0)),
            pl.BlockSpec((SENT, 1), lambda i: (0, 0)),
        ],
        out_shape=[
            jax.ShapeDtypeStruct((SENT, ANN, JEMB), jnp.float32),
            jax.ShapeDtypeStruct((SENT, ANN), jnp.float32),
            jax.ShapeDtypeStruct((SENT, 1), jnp.int32),
        ],
        scratch_shapes=[
            pltpu.VMEM((SENT, JEMB), jnp.float32),
            pltpu.VMEM((SENT, 5 * ANN), jnp.float32),
            pltpu.VMEM((5, JEMB), jnp.float32),
        ],
        interpret=interpret,
    )(obj_attn, cxt_feats, dist2, fc_w, fc7_norm_w, lfeat_norm_w, b2, cw)

    return fuse, dists, maxid[:, 0]


def kernel(cxt_feats, cxt_lfeats, obj_attn, wo_obj_idx, dist,
           fc7_norm_w, lfeat_norm_w, fc_w, fc_b):
    del wo_obj_idx
    return _run(cxt_feats, cxt_lfeats, obj_attn, dist, fc7_norm_w,
                lfeat_norm_w, fc_w, fc_b)


# speedup vs baseline: 1.4502x; 1.0360x over previous
import functools

import jax
import jax.numpy as jnp
from jax.experimental import pallas as pl
from jax.experimental.pallas import tpu as pltpu

SENT = 64
ANN = 256
FC7 = 2048
JEMB = 512
SB = 8

HI = jax.lax.Precision.HIGHEST


def _kernel(attn, cxt_feats, dist2, fc_w, w7, lw, b2, cw,
            fuse, dists, maxid, base_s, cf_s, w2_s):
    pid = pl.program_id(0)

    @pl.when(pid == 0)
    def _prologue():
        a = attn[...]
        m = jnp.max(a, axis=1, keepdims=True)
        cols = jax.lax.broadcasted_iota(jnp.int32, (SENT, ANN), 1)
        ids = jnp.min(jnp.where(a == m, cols, ANN), axis=1,
                      keepdims=True)
        maxid[...] = ids
        onehot = (cols == ids).astype(jnp.float32)
        ok = jnp.where(m == 0.0, 0.0, 1.0)

        dg = jax.lax.dot_general(onehot, dist2[...],
                                 (((1,), (1,)), ((), ())), precision=HI)
        dists[...] = jnp.where(ok == 0.0, 100.0, dg)

        w2_s[...] = jnp.transpose(fc_w[:, FC7:FC7 + 5] * lw[...]) \
            .reshape(5, JEMB)

        g = jax.lax.dot(onehot, cw[...], precision=HI)
        ss = g[:, :ANN] * g[:, :ANN]
        for c in range(1, 5):
            lc = g[:, c * ANN:(c + 1) * ANN]
            ss = ss + lc * lc
        invl = ok / jnp.maximum(jnp.sqrt(ss), 1e-12)
        cf_s[...] = g * jnp.concatenate([invl] * 5, axis=1)

        rf = jax.lax.dot(onehot, cxt_feats[...], precision=HI)
        n = jnp.sqrt(jnp.sum(rf * rf, axis=1, keepdims=True))
        inv7 = ok / jnp.maximum(n, 1e-12)
        rfn = rf * inv7 * w7[...]
        base_s[...] = jax.lax.dot_general(rfn, fc_w[:, :FC7],
                                          (((1,), (1,)), ((), ()))) + b2[...]

    sl = pl.ds(pid * SB, SB)
    cf = cf_s[sl, :]
    w2 = w2_s[...]
    baseb = base_s[sl, :][:, None, :]
    CH = 8
    for j in range(0, ANN, CH):
        acc = jnp.broadcast_to(baseb, (SB, CH, JEMB))
        for c in range(5):
            acc = acc + cf[:, c * ANN + j:c * ANN + j + CH][:, :, None] \
                * w2[c][None, None, :]
        fuse[:, j:j + CH, :] = acc


@functools.partial(jax.jit, static_argnames=("interpret",))
def _run(cxt_feats, cxt_lfeats, obj_attn, dist, fc7_norm_w, lfeat_norm_w,
         fc_w, fc_b, interpret=False):
    cw = jnp.transpose(cxt_lfeats, (1, 2, 0)).reshape(ANN, 5 * ANN)
    dist2 = dist.reshape(ANN, ANN)
    b2 = fc_b.reshape(1, JEMB)

    fuse, dists, maxid = pl.pallas_call(
        _kernel,
        grid=(SENT // SB,),
        in_specs=[
            pl.BlockSpec((SENT, ANN), lambda i: (0, 0)),
            pl.BlockSpec((ANN, FC7), lambda i: (0, 0)),
            pl.BlockSpec((ANN, ANN), lambda i: (0, 0)),
            pl.BlockSpec((JEMB, FC7 + 5), lambda i: (0, 0)),
            pl.BlockSpec((1, FC7), lambda i: (0, 0)),
            pl.BlockSpec((1, 5), lambda i: (0, 0)),
            pl.BlockSpec((1, JEMB), lambda i: (0, 0)),
            pl.BlockSpec((ANN, 5 * ANN), lambda i: (0, 0)),
        ],
        out_specs=[
            pl.BlockSpec((SB, ANN, JEMB), lambda i: (i, 0, 0)),
            pl.BlockSpec((SENT, ANN), lambda i: (0, 0)),
            pl.BlockSpec((SENT, 1), lambda i: (0, 0)),
        ],
        out_shape=[
            jax.ShapeDtypeStruct((SENT, ANN, JEMB), jnp.float32),
            jax.ShapeDtypeStruct((SENT, ANN), jnp.float32),
            jax.ShapeDtypeStruct((SENT, 1), jnp.int32),
        ],
        scratch_shapes=[
            pltpu.VMEM((SENT, JEMB), jnp.float32),
            pltpu.VMEM((SENT, 5 * ANN), jnp.float32),
            pltpu.VMEM((5, JEMB), jnp.float32),
        ],
        interpret=interpret,
    )(obj_attn, cxt_feats, dist2, fc_w, fc7_norm_w, lfeat_norm_w, b2, cw)

    return fuse, dists, maxid[:, 0]


def kernel(cxt_feats, cxt_lfeats, obj_attn, wo_obj_idx, dist,
           fc7_norm_w, lfeat_norm_w, fc_w, fc_b):
    del wo_obj_idx
    return _run(cxt_feats, cxt_lfeats, obj_attn, dist, fc7_norm_w,
                lfeat_norm_w, fc_w, fc_b)
